# baseline (speedup 1.0000x reference)
import functools
import numpy as np
import jax
import jax.numpy as jnp
from jax import lax
from jax.experimental import pallas as pl
from jax.experimental.pallas import tpu as pltpu
from jax.experimental.pallas import tpu_sc as plsc

N = 10000
F_IN = 128
HID = 64
NCLS = 40
K = 6
E = 320000

NC = 2
NS = 16
NW = NC * NS
CB = 128
ROWS_PW = 80
EPAD = NW * ROWS_PW * CB
N_ACC = 10240
TROWS = N_ACC // NS
BLK = 1000


def _worker_id():
    return lax.axis_index("s") * NC + lax.axis_index("c")


def _deg_kernel(src_hbm, dst_hbm, out_hbm, dst_v, ones_v, vbuf, acc_sh, sem):
    c = lax.axis_index("c")
    s = lax.axis_index("s")
    w = _worker_id()
    row0 = s * TROWS
    for i in range(CB // 16):
        ones_v[pl.ds(i * 16, 16)] = jnp.full((16,), 1.0, jnp.float32)
    for i in range(TROWS // 16):
        vbuf[pl.ds(i * 16, 16)] = jnp.zeros((16,), jnp.float32)
    pltpu.sync_copy(vbuf, acc_sh.at[pl.ds(row0, TROWS)])
    plsc.subcore_barrier()
    pltpu.sync_copy(dst_hbm.at[pl.ds(w * ROWS_PW, ROWS_PW)], dst_v)

    def body(j, carry):
        pltpu.sync_copy(ones_v, acc_sh.at[dst_v.at[j]], add=True)
        return carry

    lax.fori_loop(0, ROWS_PW, body, 0)
    plsc.subcore_barrier()
    pltpu.sync_copy(acc_sh.at[pl.ds(row0, TROWS)], vbuf)
    pltpu.sync_copy(vbuf, out_hbm.at[c, pl.ds(row0, TROWS)])


def _ring_pass(g_hbm, src_v, dst_v, bufs, acc_sh, sem):
    b0, b1, b2, b3 = bufs

    def start_g(buf, j):
        pltpu.async_copy(g_hbm.at[src_v.at[j]], buf, sem)

    def wait_g2(x, y):
        pltpu.make_async_copy(g_hbm.at[pl.ds(0, CB)], x, sem).wait()
        pltpu.make_async_copy(g_hbm.at[pl.ds(0, CB)], y, sem).wait()

    def scat(buf, j):
        pltpu.sync_copy(buf, acc_sh.at[dst_v.at[j]], add=True)

    NH = ROWS_PW // 4
    start_g(b0, 0)
    start_g(b1, 1)

    def body(h, carry):
        j = h * 4
        wait_g2(b0, b1)
        start_g(b2, j + 2)
        start_g(b3, j + 3)
        scat(b0, j)
        scat(b1, j + 1)
        wait_g2(b2, b3)

        @pl.when(h + 1 < NH)
        def _():
            start_g(b0, j + 4)
            start_g(b1, j + 5)

        scat(b2, j + 2)
        scat(b3, j + 3)
        return carry

    lax.fori_loop(0, NH, body, 0)


def _prop_kernel(D, nt, src_hbm, dst_hbm, *args):
    tables = args[:nt]
    zeros_hbm = args[nt]
    outs = args[nt + 1:2 * nt + 1]
    src_v, dst_v, b0, b1, b2, b3, acc_sh, sem = args[2 * nt + 1:]
    c = lax.axis_index("c")
    s = lax.axis_index("s")
    w = _worker_id()
    row0 = s * TROWS
    pltpu.sync_copy(zeros_hbm, b0)
    for i in range(TROWS // CB):
        pltpu.sync_copy(b0, acc_sh.at[pl.ds(row0 + i * CB, CB)])
    pltpu.sync_copy(src_hbm.at[pl.ds(w * ROWS_PW, ROWS_PW)], src_v)
    pltpu.sync_copy(dst_hbm.at[pl.ds(w * ROWS_PW, ROWS_PW)], dst_v)
    plsc.subcore_barrier()
    for t in range(nt):
        _ring_pass(tables[t], src_v, dst_v, (b0, b1, b2, b3), acc_sh, sem)
        plsc.subcore_barrier()
        for i in range(TROWS // CB):
            pltpu.sync_copy(acc_sh.at[pl.ds(row0 + i * CB, CB)], b0)
            pltpu.sync_copy(b0, outs[t].at[c, pl.ds(row0 + i * CB, CB)])
        if t + 1 < nt:
            pltpu.sync_copy(zeros_hbm, b0)
            for i in range(TROWS // CB):
                pltpu.sync_copy(b0, acc_sh.at[pl.ds(row0 + i * CB, CB)])
            plsc.subcore_barrier()


_SC_PARAMS = pltpu.CompilerParams(use_tc_tiling_on_sc=False)


def _make_deg():
    mesh = plsc.VectorSubcoreMesh(core_axis_name="c", subcore_axis_name="s")
    return functools.partial(
        pl.kernel, _deg_kernel, mesh=mesh,
        compiler_params=_SC_PARAMS,
        out_type=jax.ShapeDtypeStruct((NC, N_ACC), jnp.float32),
        scratch_types=[
            pltpu.VMEM((ROWS_PW, CB), jnp.int32),
            pltpu.VMEM((CB,), jnp.float32),
            pltpu.VMEM((TROWS,), jnp.float32),
            pltpu.VMEM_SHARED((N_ACC,), jnp.float32),
            pltpu.SemaphoreType.DMA,
        ],
    )()


def _make_prop(D, nt=1):
    mesh = plsc.VectorSubcoreMesh(core_axis_name="c", subcore_axis_name="s")
    return functools.partial(
        pl.kernel, functools.partial(_prop_kernel, D, nt), mesh=mesh,
        compiler_params=_SC_PARAMS,
        out_type=[jax.ShapeDtypeStruct((NC, N_ACC, D), jnp.float32)
                  for _ in range(nt)],
        scratch_types=[
            pltpu.VMEM((ROWS_PW, CB), jnp.int32),
            pltpu.VMEM((ROWS_PW, CB), jnp.int32),
            pltpu.VMEM((CB, D), jnp.float32),
            pltpu.VMEM((CB, D), jnp.float32),
            pltpu.VMEM((CB, D), jnp.float32),
            pltpu.VMEM((CB, D), jnp.float32),
            pltpu.VMEM_SHARED((N_ACC, D), jnp.float32),
            pltpu.SemaphoreType.DMA,
        ],
    )()


def _dinv(degp_v):
    deg = degp_v[0] + degp_v[1] + 1.0
    return lax.rsqrt(jnp.maximum(deg, 1.0))


def _dense1_body(x_ref, degp_ref, w1_ref, b1_ref, means_ref, logvars_ref,
                 logp_ref, g0a_ref, g0b_ref, g0c_ref, lg_ref):
    x = x_ref[...]
    dinv = _dinv(degp_ref[...])
    isn = x != x
    m = jnp.where(isn, 1.0, 0.0)
    xc = jnp.where(isn, 0.0, x)
    a = jax.lax.dot_general(xc, w1_ref[...], (((1,), (0,)), ((), ())),
                            preferred_element_type=jnp.float32) + b1_ref[...]
    g0a_ref[...] = dinv * a
    dm = dinv * m
    g0b_ref[...] = dm[:, :HID]
    g0c_ref[...] = dm[:, HID:]
    means = means_ref[...]
    logvars = logvars_ref[...]
    var = jnp.exp(logvars)
    p1 = -0.5 / var
    p2 = means / var
    p3 = -0.5 * means * means / var
    dn = (((1,), (1,)), ((), ()))
    lg = (jax.lax.dot_general(xc * xc, p1, dn, preferred_element_type=jnp.float32)
          + jax.lax.dot_general(xc, p2, dn, preferred_element_type=jnp.float32)
          + jax.lax.dot_general(1.0 - m, p3, dn, preferred_element_type=jnp.float32))
    const = (logp_ref[...] - 0.5 * jnp.sum(logvars, axis=1)[None, :]
             - 0.5 * F_IN * np.log(2.0 * np.pi))
    kidx = lax.broadcasted_iota(jnp.int32, lg.shape, 1)
    lg_ref[...] = jnp.where(kidx < K, lg + const, -1e30)


def _dense2_body(s1a_ref, s1b_ref, s1c_ref, g0a_ref, g0b_ref, g0c_ref,
                 degp_ref, lg_ref, w1_ref, means_ref, logvars_ref, w2_ref,
                 g1_ref):
    dinv = _dinv(degp_ref[...])
    s1a = s1a_ref[...]
    s1b = s1b_ref[...]
    s1c = s1c_ref[...]
    pa = dinv * (s1a[0] + s1a[1] + g0a_ref[...])
    pm = dinv * jnp.concatenate(
        [s1b[0] + s1b[1] + g0b_ref[...],
         s1c[0] + s1c[1] + g0c_ref[...]], axis=1)
    w1 = w1_ref[...]
    w1sq = w1 * w1
    means = means_ref[...]
    var = jnp.exp(logvars_ref[...])
    lg = lg_ref[...]
    gmax = jnp.max(lg, axis=1, keepdims=True)
    ge = jnp.exp(lg - gmax)
    gamma = ge / jnp.sum(ge, axis=1, keepdims=True)
    dn = (((1,), (0,)), ((), ()))
    x1 = jnp.zeros((BLK, HID), jnp.float32)
    for k in range(K):
        mk = means[k:k + 1, :]
        vk = var[k:k + 1, :]
        tx = pa + jax.lax.dot_general(pm * mk, w1, dn,
                                      preferred_element_type=jnp.float32)
        tc = jax.lax.dot_general(pm * vk, w1sq, dn,
                                 preferred_element_type=jnp.float32)
        sig = jnp.sqrt(tc + 1e-8)
        ratio = tx / sig
        cdf = 0.5 * (1.0 + lax.erf(ratio * np.float32(1.0 / np.sqrt(2.0))))
        pdf = jnp.exp(-0.5 * ratio * ratio) * np.float32(1.0 / np.sqrt(2.0 * np.pi))
        x1 = x1 + gamma[:, k:k + 1] * (tx * cdf + sig * pdf)
    g1_ref[...] = dinv * jax.lax.dot_general(x1, w2_ref[...], dn,
                                             preferred_element_type=jnp.float32)


def _dense3_body(s2_ref, g1_ref, degp_ref, b2_ref, w3_ref, x2_ref, g2_ref):
    dinv = _dinv(degp_ref[...])
    s2 = s2_ref[...]
    x2 = jnp.maximum(dinv * (s2[0] + s2[1] + g1_ref[...]) + b2_ref[...], 0.0)
    x2_ref[...] = x2
    g2_ref[...] = dinv * jax.lax.dot_general(
        x2, w3_ref[...], (((1,), (0,)), ((), ())),
        preferred_element_type=jnp.float32)


def _dense4_body(s3_ref, g2_ref, degp_ref, b3_ref, out_ref):
    dinv = _dinv(degp_ref[...])
    s3 = s3_ref[...]
    x3 = dinv * (s3[0] + s3[1] + g2_ref[...]) + b3_ref[...]
    cidx = lax.broadcasted_iota(jnp.int32, x3.shape, 1)
    x3 = jnp.where(cidx < NCLS, x3, -1e30)
    m = jnp.max(x3, axis=1, keepdims=True)
    z = x3 - m
    lse = jnp.log(jnp.sum(jnp.where(cidx < NCLS, jnp.exp(z), 0.0),
                          axis=1, keepdims=True))
    out_ref[...] = (z - lse)[:, :NCLS]


def _row_spec(d):
    return pl.BlockSpec((BLK, d), lambda i: (i, 0))


def _part_spec(d):
    return pl.BlockSpec((NC, BLK, d), lambda i: (0, i, 0))


def _full_spec(shape):
    nd = len(shape)
    return pl.BlockSpec(shape, lambda i: (0,) * nd)


def kernel(x, edge_index, logp, means, logvars, W1, b1, W2, b2, W3, b3):
    f32 = jnp.float32
    pad_i = jnp.arange(EPAD - E, dtype=jnp.int32)
    src = jnp.concatenate([edge_index[0], (pad_i * 97) % N])
    dst = jnp.concatenate([edge_index[1], N + pad_i % (N_ACC - N)])
    src2 = src.reshape(NW * ROWS_PW, CB)
    dst2 = dst.reshape(NW * ROWS_PW, CB)
    means8 = jnp.concatenate([means, jnp.zeros((8 - K, F_IN), f32)], axis=0)
    logvars8 = jnp.concatenate([logvars, jnp.zeros((8 - K, F_IN), f32)], axis=0)
    logp8 = jnp.concatenate([logp, jnp.zeros((8 - K,), f32)]).reshape(1, 8)
    b1r = b1.reshape(1, HID)
    b2r = b2.reshape(1, HID)
    b3r = jnp.concatenate([b3, jnp.zeros((8,), f32)]).reshape(1, NCLS + 8)
    w3p = jnp.concatenate([W3, jnp.zeros((HID, 8), f32)], axis=1)
    z64 = jnp.zeros((CB, HID), f32)
    z48 = jnp.zeros((CB, NCLS + 8), f32)

    grid = N // BLK

    degp = _make_deg()(src2, dst2)
    degp3 = degp.reshape(NC, N_ACC, 1)

    g0a, g0b, g0c, lg = pl.pallas_call(
        _dense1_body,
        grid=(grid,),
        in_specs=[_row_spec(F_IN), _part_spec(1), _full_spec((F_IN, HID)),
                  _full_spec((1, HID)), _full_spec((8, F_IN)),
                  _full_spec((8, F_IN)), _full_spec((1, 8))],
        out_specs=[_row_spec(HID), _row_spec(HID), _row_spec(HID),
                   _row_spec(8)],
        out_shape=[jax.ShapeDtypeStruct((N, HID), f32),
                   jax.ShapeDtypeStruct((N, HID), f32),
                   jax.ShapeDtypeStruct((N, HID), f32),
                   jax.ShapeDtypeStruct((N, 8), f32)],
    )(x, degp3, W1, b1r, means8, logvars8, logp8)

    s1a, s1b, s1c = _make_prop(HID, nt=3)(src2, dst2, g0a, g0b, g0c, z64)

    g1 = pl.pallas_call(
        _dense2_body,
        grid=(grid,),
        in_specs=[_part_spec(HID), _part_spec(HID), _part_spec(HID),
                  _row_spec(HID), _row_spec(HID), _row_spec(HID),
                  _part_spec(1),
                  _row_spec(8), _full_spec((F_IN, HID)), _full_spec((8, F_IN)),
                  _full_spec((8, F_IN)), _full_spec((HID, HID))],
        out_specs=[_row_spec(HID)],
        out_shape=[jax.ShapeDtypeStruct((N, HID), f32)],
    )(s1a, s1b, s1c, g0a, g0b, g0c, degp3, lg, W1, means8, logvars8, W2)[0]

    s2 = _make_prop(HID)(src2, dst2, g1, z64)[0]

    x2, g2 = pl.pallas_call(
        _dense3_body,
        grid=(grid,),
        in_specs=[_part_spec(HID), _row_spec(HID), _part_spec(1),
                  _full_spec((1, HID)), _full_spec((HID, NCLS + 8))],
        out_specs=[_row_spec(HID), _row_spec(NCLS + 8)],
        out_shape=[jax.ShapeDtypeStruct((N, HID), f32),
                   jax.ShapeDtypeStruct((N, NCLS + 8), f32)],
    )(s2, g1, degp3, b2r, w3p)

    s3 = _make_prop(NCLS + 8)(src2, dst2, g2, z48)[0]

    out1 = pl.pallas_call(
        _dense4_body,
        grid=(grid,),
        in_specs=[_part_spec(NCLS + 8), _row_spec(NCLS + 8), _part_spec(1),
                  _full_spec((1, NCLS + 8))],
        out_specs=[_row_spec(NCLS)],
        out_shape=[jax.ShapeDtypeStruct((N, NCLS), f32)],
    )(s3, g2, degp3, b3r)[0]

    return out1, x2


# speedup vs baseline: 1.0181x; 1.0181x over previous
import functools
import numpy as np
import jax
import jax.numpy as jnp
from jax import lax
from jax.experimental import pallas as pl
from jax.experimental.pallas import tpu as pltpu
from jax.experimental.pallas import tpu_sc as plsc

N = 10000
F_IN = 128
HID = 64
NCLS = 40
K = 6
E = 320000

NC = 2
NS = 16
NW = NC * NS
CB = 128
ROWS_PW = 80
EPAD = NW * ROWS_PW * CB
N_ACC = 10240
TROWS = N_ACC // NS
BLK = 1000


def _worker_id():
    return lax.axis_index("s") * NC + lax.axis_index("c")


def _deg_kernel(src_hbm, dst_hbm, out_hbm, dst_v, ones_v, vbuf, acc_sh, sem):
    c = lax.axis_index("c")
    s = lax.axis_index("s")
    w = _worker_id()
    row0 = s * TROWS
    for i in range(CB // 16):
        ones_v[pl.ds(i * 16, 16)] = jnp.full((16,), 1.0, jnp.float32)
    for i in range(TROWS // 16):
        vbuf[pl.ds(i * 16, 16)] = jnp.zeros((16,), jnp.float32)
    pltpu.sync_copy(vbuf, acc_sh.at[pl.ds(row0, TROWS)])
    plsc.subcore_barrier()
    pltpu.sync_copy(dst_hbm.at[pl.ds(w * ROWS_PW, ROWS_PW)], dst_v)

    def body(j, carry):
        pltpu.sync_copy(ones_v, acc_sh.at[dst_v.at[j]], add=True)
        return carry

    lax.fori_loop(0, ROWS_PW, body, 0)
    plsc.subcore_barrier()
    pltpu.sync_copy(acc_sh.at[pl.ds(row0, TROWS)], vbuf)
    pltpu.sync_copy(vbuf, out_hbm.at[c, pl.ds(row0, TROWS)])


def _ring_pass(g_hbm, src_v, dst_v, bufs, acc_sh, sem):
    b0, b1, b2, b3 = bufs

    def start_g(buf, j):
        pltpu.async_copy(g_hbm.at[src_v.at[j]], buf, sem)

    def wait_g2(x, y):
        pltpu.make_async_copy(g_hbm.at[pl.ds(0, CB)], x, sem).wait()
        pltpu.make_async_copy(g_hbm.at[pl.ds(0, CB)], y, sem).wait()

    def scat(buf, j):
        pltpu.sync_copy(buf, acc_sh.at[dst_v.at[j]], add=True)

    NH = ROWS_PW // 4
    start_g(b0, 0)
    start_g(b1, 1)

    def body(h, carry):
        j = h * 4
        wait_g2(b0, b1)
        start_g(b2, j + 2)
        start_g(b3, j + 3)
        scat(b0, j)
        scat(b1, j + 1)
        wait_g2(b2, b3)

        @pl.when(h + 1 < NH)
        def _():
            start_g(b0, j + 4)
            start_g(b1, j + 5)

        scat(b2, j + 2)
        scat(b3, j + 3)
        return carry

    lax.fori_loop(0, NH, body, 0)


def _prop_kernel(D, nt, src_hbm, dst_hbm, *args):
    tables = args[:nt]
    zeros_hbm = args[nt]
    outs = args[nt + 1:2 * nt + 1]
    src_v, dst_v, b0, b1, b2, b3, acc_sh, sem = args[2 * nt + 1:]
    c = lax.axis_index("c")
    s = lax.axis_index("s")
    w = _worker_id()
    row0 = s * TROWS
    pltpu.sync_copy(zeros_hbm, b0)
    for i in range(TROWS // CB):
        pltpu.sync_copy(b0, acc_sh.at[pl.ds(row0 + i * CB, CB)])
    pltpu.sync_copy(src_hbm.at[pl.ds(w * ROWS_PW, ROWS_PW)], src_v)
    pltpu.sync_copy(dst_hbm.at[pl.ds(w * ROWS_PW, ROWS_PW)], dst_v)
    plsc.subcore_barrier()
    for t in range(nt):
        _ring_pass(tables[t], src_v, dst_v, (b0, b1, b2, b3), acc_sh, sem)
        plsc.subcore_barrier()
        for i in range(TROWS // CB):
            pltpu.sync_copy(acc_sh.at[pl.ds(row0 + i * CB, CB)], b0)
            pltpu.sync_copy(b0, outs[t].at[c, pl.ds(row0 + i * CB, CB)])
        if t + 1 < nt:
            pltpu.sync_copy(zeros_hbm, b0)
            for i in range(TROWS // CB):
                pltpu.sync_copy(b0, acc_sh.at[pl.ds(row0 + i * CB, CB)])
            plsc.subcore_barrier()


_SC_PARAMS = pltpu.CompilerParams(use_tc_tiling_on_sc=False)


def _make_deg():
    mesh = plsc.VectorSubcoreMesh(core_axis_name="c", subcore_axis_name="s")
    return functools.partial(
        pl.kernel, _deg_kernel, mesh=mesh,
        compiler_params=_SC_PARAMS,
        out_type=jax.ShapeDtypeStruct((NC, N_ACC), jnp.float32),
        scratch_types=[
            pltpu.VMEM((ROWS_PW, CB), jnp.int32),
            pltpu.VMEM((CB,), jnp.float32),
            pltpu.VMEM((TROWS,), jnp.float32),
            pltpu.VMEM_SHARED((N_ACC,), jnp.float32),
            pltpu.SemaphoreType.DMA,
        ],
    )()


def _make_prop(D, nt=1):
    mesh = plsc.VectorSubcoreMesh(core_axis_name="c", subcore_axis_name="s")
    return functools.partial(
        pl.kernel, functools.partial(_prop_kernel, D, nt), mesh=mesh,
        compiler_params=_SC_PARAMS,
        out_type=[jax.ShapeDtypeStruct((NC, N_ACC, D), jnp.float32)
                  for _ in range(nt)],
        scratch_types=[
            pltpu.VMEM((ROWS_PW, CB), jnp.int32),
            pltpu.VMEM((ROWS_PW, CB), jnp.int32),
            pltpu.VMEM((CB, D), jnp.float32),
            pltpu.VMEM((CB, D), jnp.float32),
            pltpu.VMEM((CB, D), jnp.float32),
            pltpu.VMEM((CB, D), jnp.float32),
            pltpu.VMEM_SHARED((N_ACC, D), jnp.float32),
            pltpu.SemaphoreType.DMA,
        ],
    )()


def _dinv(degp_v):
    deg = degp_v[0] + degp_v[1] + 1.0
    return lax.rsqrt(jnp.maximum(deg, 1.0))


def _dense1_body(x_ref, degp_ref, w1_ref, b1_ref, means_ref, logvars_ref,
                 logp_ref, g0a_ref, g0b_ref, g0c_ref, lg_ref):
    x = x_ref[...]
    dinv = _dinv(degp_ref[...])
    isn = x != x
    m = jnp.where(isn, 1.0, 0.0)
    xc = jnp.where(isn, 0.0, x)
    a = jax.lax.dot_general(xc, w1_ref[...], (((1,), (0,)), ((), ())),
                            preferred_element_type=jnp.float32) + b1_ref[...]
    g0a_ref[...] = dinv * a
    dm = dinv * m
    g0b_ref[...] = dm[:, :HID]
    g0c_ref[...] = dm[:, HID:]
    means = means_ref[...]
    logvars = logvars_ref[...]
    var = jnp.exp(logvars)
    p1 = -0.5 / var
    p2 = means / var
    p3 = -0.5 * means * means / var
    dn = (((1,), (1,)), ((), ()))
    lg = (jax.lax.dot_general(xc * xc, p1, dn, preferred_element_type=jnp.float32)
          + jax.lax.dot_general(xc, p2, dn, preferred_element_type=jnp.float32)
          + jax.lax.dot_general(1.0 - m, p3, dn, preferred_element_type=jnp.float32))
    const = (logp_ref[...] - 0.5 * jnp.sum(logvars, axis=1)[None, :]
             - 0.5 * F_IN * np.log(2.0 * np.pi))
    kidx = lax.broadcasted_iota(jnp.int32, lg.shape, 1)
    lg_ref[...] = jnp.where(kidx < K, lg + const, -1e30)


def _dense2_body(s1a_ref, s1b_ref, s1c_ref, g0a_ref, g0b_ref, g0c_ref,
                 degp_ref, lg_ref, w1_ref, meansT_ref, logvarsT_ref, w2_ref,
                 g1_ref):
    dinv = _dinv(degp_ref[...])
    s1a = s1a_ref[...]
    s1b = s1b_ref[...]
    s1c = s1c_ref[...]
    pa = dinv * (s1a[0] + s1a[1] + g0a_ref[...])
    pm = dinv * jnp.concatenate(
        [s1b[0] + s1b[1] + g0b_ref[...],
         s1c[0] + s1c[1] + g0c_ref[...]], axis=1)
    w1 = w1_ref[...]
    w1sq = w1 * w1
    meansT = meansT_ref[...]
    varT = jnp.exp(logvarsT_ref[...])
    lg = lg_ref[...]
    gmax = jnp.max(lg, axis=1, keepdims=True)
    ge = jnp.exp(lg - gmax)
    gamma = ge / jnp.sum(ge, axis=1, keepdims=True)
    dn = (((1,), (0,)), ((), ()))
    wm = jnp.concatenate([meansT[:, k:k + 1] * w1
                          for k in range(K)], axis=1)
    tx = jax.lax.dot_general(pm, wm, dn, preferred_element_type=jnp.float32)
    wv = jnp.concatenate([varT[:, k:k + 1] * w1sq
                          for k in range(K)], axis=1)
    tc = jax.lax.dot_general(pm, wv, dn, preferred_element_type=jnp.float32)
    pa6 = jnp.concatenate([pa] * K, axis=1)
    tx = tx + pa6
    tce = tc + 1e-8
    rs = lax.rsqrt(tce)
    ratio = tx * rs
    cdf = 0.5 * (1.0 + lax.erf(ratio * np.float32(1.0 / np.sqrt(2.0))))
    pdf = jnp.exp(-0.5 * ratio * ratio) * np.float32(1.0 / np.sqrt(2.0 * np.pi))
    er = tx * cdf + tce * rs * pdf
    kl = lax.broadcasted_iota(jnp.int32, (8, K * HID), 1) // HID
    kr = lax.broadcasted_iota(jnp.int32, (8, K * HID), 0)
    rep = jnp.where(kl == kr, 1.0, 0.0)
    gexp = jax.lax.dot_general(gamma, rep, dn,
                               preferred_element_type=jnp.float32)
    ge_r = gexp * er
    x1 = ge_r[:, 0:HID]
    for k in range(1, K):
        x1 = x1 + ge_r[:, k * HID:(k + 1) * HID]
    g1_ref[...] = dinv * jax.lax.dot_general(x1, w2_ref[...], dn,
                                             preferred_element_type=jnp.float32)


def _dense3_body(s2_ref, g1_ref, degp_ref, b2_ref, w3_ref, x2_ref, g2_ref):
    dinv = _dinv(degp_ref[...])
    s2 = s2_ref[...]
    x2 = jnp.maximum(dinv * (s2[0] + s2[1] + g1_ref[...]) + b2_ref[...], 0.0)
    x2_ref[...] = x2
    g2_ref[...] = dinv * jax.lax.dot_general(
        x2, w3_ref[...], (((1,), (0,)), ((), ())),
        preferred_element_type=jnp.float32)


def _dense4_body(s3_ref, g2_ref, degp_ref, b3_ref, out_ref):
    dinv = _dinv(degp_ref[...])
    s3 = s3_ref[...]
    x3 = dinv * (s3[0] + s3[1] + g2_ref[...]) + b3_ref[...]
    cidx = lax.broadcasted_iota(jnp.int32, x3.shape, 1)
    x3 = jnp.where(cidx < NCLS, x3, -1e30)
    m = jnp.max(x3, axis=1, keepdims=True)
    z = x3 - m
    lse = jnp.log(jnp.sum(jnp.where(cidx < NCLS, jnp.exp(z), 0.0),
                          axis=1, keepdims=True))
    out_ref[...] = (z - lse)[:, :NCLS]


def _row_spec(d):
    return pl.BlockSpec((BLK, d), lambda i: (i, 0))


def _part_spec(d):
    return pl.BlockSpec((NC, BLK, d), lambda i: (0, i, 0))


def _full_spec(shape):
    nd = len(shape)
    return pl.BlockSpec(shape, lambda i: (0,) * nd)


def kernel(x, edge_index, logp, means, logvars, W1, b1, W2, b2, W3, b3):
    f32 = jnp.float32
    pad_i = jnp.arange(EPAD - E, dtype=jnp.int32)
    src = jnp.concatenate([edge_index[0], (pad_i * 97) % N])
    dst = jnp.concatenate([edge_index[1], N + pad_i % (N_ACC - N)])
    src2 = src.reshape(NW * ROWS_PW, CB)
    dst2 = dst.reshape(NW * ROWS_PW, CB)
    means8 = jnp.concatenate([means, jnp.zeros((8 - K, F_IN), f32)], axis=0)
    logvars8 = jnp.concatenate([logvars, jnp.zeros((8 - K, F_IN), f32)], axis=0)
    logp8 = jnp.concatenate([logp, jnp.zeros((8 - K,), f32)]).reshape(1, 8)
    b1r = b1.reshape(1, HID)
    b2r = b2.reshape(1, HID)
    b3r = jnp.concatenate([b3, jnp.zeros((8,), f32)]).reshape(1, NCLS + 8)
    w3p = jnp.concatenate([W3, jnp.zeros((HID, 8), f32)], axis=1)
    z64 = jnp.zeros((CB, HID), f32)
    z48 = jnp.zeros((CB, NCLS + 8), f32)

    grid = N // BLK

    degp = _make_deg()(src2, dst2)
    degp3 = degp.reshape(NC, N_ACC, 1)

    g0a, g0b, g0c, lg = pl.pallas_call(
        _dense1_body,
        grid=(grid,),
        in_specs=[_row_spec(F_IN), _part_spec(1), _full_spec((F_IN, HID)),
                  _full_spec((1, HID)), _full_spec((8, F_IN)),
                  _full_spec((8, F_IN)), _full_spec((1, 8))],
        out_specs=[_row_spec(HID), _row_spec(HID), _row_spec(HID),
                   _row_spec(8)],
        out_shape=[jax.ShapeDtypeStruct((N, HID), f32),
                   jax.ShapeDtypeStruct((N, HID), f32),
                   jax.ShapeDtypeStruct((N, HID), f32),
                   jax.ShapeDtypeStruct((N, 8), f32)],
    )(x, degp3, W1, b1r, means8, logvars8, logp8)

    s1a, s1b, s1c = _make_prop(HID, nt=3)(src2, dst2, g0a, g0b, g0c, z64)

    g1 = pl.pallas_call(
        _dense2_body,
        grid=(grid,),
        in_specs=[_part_spec(HID), _part_spec(HID), _part_spec(HID),
                  _row_spec(HID), _row_spec(HID), _row_spec(HID),
                  _part_spec(1),
                  _row_spec(8), _full_spec((F_IN, HID)), _full_spec((F_IN, 8)),
                  _full_spec((F_IN, 8)), _full_spec((HID, HID))],
        out_specs=[_row_spec(HID)],
        out_shape=[jax.ShapeDtypeStruct((N, HID), f32)],
    )(s1a, s1b, s1c, g0a, g0b, g0c, degp3, lg, W1, means8.T,
      logvars8.T, W2)[0]

    s2 = _make_prop(HID)(src2, dst2, g1, z64)[0]

    x2, g2 = pl.pallas_call(
        _dense3_body,
        grid=(grid,),
        in_specs=[_part_spec(HID), _row_spec(HID), _part_spec(1),
                  _full_spec((1, HID)), _full_spec((HID, NCLS + 8))],
        out_specs=[_row_spec(HID), _row_spec(NCLS + 8)],
        out_shape=[jax.ShapeDtypeStruct((N, HID), f32),
                   jax.ShapeDtypeStruct((N, NCLS + 8), f32)],
    )(s2, g1, degp3, b2r, w3p)

    s3 = _make_prop(NCLS + 8)(src2, dst2, g2, z48)[0]

    out1 = pl.pallas_call(
        _dense4_body,
        grid=(grid,),
        in_specs=[_part_spec(NCLS + 8), _row_spec(NCLS + 8), _part_spec(1),
                  _full_spec((1, NCLS + 8))],
        out_specs=[_row_spec(NCLS)],
        out_shape=[jax.ShapeDtypeStruct((N, NCLS), f32)],
    )(s3, g2, degp3, b3r)[0]

    return out1, x2


# speedup vs baseline: 1.0366x; 1.0182x over previous
import functools
import numpy as np
import jax
import jax.numpy as jnp
from jax import lax
from jax.experimental import pallas as pl
from jax.experimental.pallas import tpu as pltpu
from jax.experimental.pallas import tpu_sc as plsc

N = 10000
F_IN = 128
HID = 64
NCLS = 40
K = 6
E = 320000

NC = 2
NS = 16
NW = NC * NS
CB = 128
ROWS_PW = 80
EPAD = NW * ROWS_PW * CB
N_ACC = 10240
TROWS = N_ACC // NS
BLK = 1024


def _worker_id():
    return lax.axis_index("s") * NC + lax.axis_index("c")


def _deg_kernel(src_hbm, dst_hbm, out_hbm, dst_v, ones_v, vbuf, acc_sh, sem):
    c = lax.axis_index("c")
    s = lax.axis_index("s")
    w = _worker_id()
    row0 = s * TROWS
    for i in range(CB // 16):
        ones_v[pl.ds(i * 16, 16)] = jnp.full((16,), 1.0, jnp.float32)
    for i in range(TROWS // 16):
        vbuf[pl.ds(i * 16, 16)] = jnp.zeros((16,), jnp.float32)
    pltpu.sync_copy(vbuf, acc_sh.at[pl.ds(row0, TROWS)])
    plsc.subcore_barrier()
    pltpu.sync_copy(dst_hbm.at[pl.ds(w * ROWS_PW, ROWS_PW)], dst_v)

    def body(j, carry):
        pltpu.sync_copy(ones_v, acc_sh.at[dst_v.at[j]], add=True)
        return carry

    lax.fori_loop(0, ROWS_PW, body, 0)
    plsc.subcore_barrier()
    pltpu.sync_copy(acc_sh.at[pl.ds(row0, TROWS)], vbuf)
    pltpu.sync_copy(vbuf, out_hbm.at[c, 0, pl.ds(row0, TROWS)])


def _ring_pass(g_hbm, src_v, dst_v, bufs, acc_sh, sem):
    b0, b1, b2, b3 = bufs

    def start_g(buf, j):
        pltpu.async_copy(g_hbm.at[src_v.at[j]], buf, sem)

    def wait_g2(x, y):
        pltpu.make_async_copy(g_hbm.at[pl.ds(0, CB)], x, sem).wait()
        pltpu.make_async_copy(g_hbm.at[pl.ds(0, CB)], y, sem).wait()

    def scat(buf, j):
        pltpu.sync_copy(buf, acc_sh.at[dst_v.at[j]], add=True)

    NH = ROWS_PW // 4
    start_g(b0, 0)
    start_g(b1, 1)

    def body(h, carry):
        j = h * 4
        wait_g2(b0, b1)
        start_g(b2, j + 2)
        start_g(b3, j + 3)
        scat(b0, j)
        scat(b1, j + 1)
        wait_g2(b2, b3)

        @pl.when(h + 1 < NH)
        def _():
            start_g(b0, j + 4)
            start_g(b1, j + 5)

        scat(b2, j + 2)
        scat(b3, j + 3)
        return carry

    lax.fori_loop(0, NH, body, 0)


def _prop_kernel(D, nt, src_hbm, dst_hbm, *args):
    tables = args[:nt]
    zeros_hbm = args[nt]
    outs = args[nt + 1:2 * nt + 1]
    src_v, dst_v, b0, b1, b2, b3, acc_sh, sem = args[2 * nt + 1:]
    c = lax.axis_index("c")
    s = lax.axis_index("s")
    w = _worker_id()
    row0 = s * TROWS
    pltpu.sync_copy(zeros_hbm, b0)
    for i in range(TROWS // CB):
        pltpu.sync_copy(b0, acc_sh.at[pl.ds(row0 + i * CB, CB)])
    pltpu.sync_copy(src_hbm.at[pl.ds(w * ROWS_PW, ROWS_PW)], src_v)
    pltpu.sync_copy(dst_hbm.at[pl.ds(w * ROWS_PW, ROWS_PW)], dst_v)
    plsc.subcore_barrier()
    for t in range(nt):
        _ring_pass(tables[t], src_v, dst_v, (b0, b1, b2, b3), acc_sh, sem)
        plsc.subcore_barrier()
        for i in range(TROWS // CB):
            pltpu.sync_copy(acc_sh.at[pl.ds(row0 + i * CB, CB)], b0)
            pltpu.sync_copy(b0, outs[t].at[c, pl.ds(row0 + i * CB, CB)])
        if t + 1 < nt:
            pltpu.sync_copy(zeros_hbm, b0)
            for i in range(TROWS // CB):
                pltpu.sync_copy(b0, acc_sh.at[pl.ds(row0 + i * CB, CB)])
            plsc.subcore_barrier()


_SC_PARAMS = pltpu.CompilerParams(use_tc_tiling_on_sc=False)


def _make_deg():
    mesh = plsc.VectorSubcoreMesh(core_axis_name="c", subcore_axis_name="s")
    return functools.partial(
        pl.kernel, _deg_kernel, mesh=mesh,
        compiler_params=_SC_PARAMS,
        out_type=jax.ShapeDtypeStruct((NC, 1, N_ACC), jnp.float32),
        scratch_types=[
            pltpu.VMEM((ROWS_PW, CB), jnp.int32),
            pltpu.VMEM((CB,), jnp.float32),
            pltpu.VMEM((TROWS,), jnp.float32),
            pltpu.VMEM_SHARED((N_ACC,), jnp.float32),
            pltpu.SemaphoreType.DMA,
        ],
    )()


def _make_prop(D, nt=1):
    mesh = plsc.VectorSubcoreMesh(core_axis_name="c", subcore_axis_name="s")
    return functools.partial(
        pl.kernel, functools.partial(_prop_kernel, D, nt), mesh=mesh,
        compiler_params=_SC_PARAMS,
        out_type=[jax.ShapeDtypeStruct((NC, N_ACC, D), jnp.float32)
                  for _ in range(nt)],
        scratch_types=[
            pltpu.VMEM((ROWS_PW, CB), jnp.int32),
            pltpu.VMEM((ROWS_PW, CB), jnp.int32),
            pltpu.VMEM((CB, D), jnp.float32),
            pltpu.VMEM((CB, D), jnp.float32),
            pltpu.VMEM((CB, D), jnp.float32),
            pltpu.VMEM((CB, D), jnp.float32),
            pltpu.VMEM_SHARED((N_ACC, D), jnp.float32),
            pltpu.SemaphoreType.DMA,
        ],
    )()


def _dinv(degp_v):
    deg = degp_v[0] + degp_v[1] + 1.0
    return lax.rsqrt(jnp.maximum(jnp.transpose(deg, (1, 0)), 1.0))


def _dense1_body(x_ref, degp_ref, w1_ref, b1_ref, means_ref, logvars_ref,
                 logp_ref, g0a_ref, g0b_ref, g0c_ref, lg_ref):
    x = x_ref[...]
    dinv = _dinv(degp_ref[...])
    isn = x != x
    m = jnp.where(isn, 1.0, 0.0)
    xc = jnp.where(isn, 0.0, x)
    a = jax.lax.dot_general(xc, w1_ref[...], (((1,), (0,)), ((), ())),
                            preferred_element_type=jnp.float32) + b1_ref[...]
    g0a_ref[...] = dinv * a
    dm = dinv * m
    g0b_ref[...] = dm[:, :HID]
    g0c_ref[...] = dm[:, HID:]
    means = means_ref[...]
    logvars = logvars_ref[...]
    var = jnp.exp(logvars)
    p1 = -0.5 / var
    p2 = means / var
    p3 = -0.5 * means * means / var
    dn = (((1,), (1,)), ((), ()))
    lg = (jax.lax.dot_general(xc * xc, p1, dn, preferred_element_type=jnp.float32)
          + jax.lax.dot_general(xc, p2, dn, preferred_element_type=jnp.float32)
          + jax.lax.dot_general(1.0 - m, p3, dn, preferred_element_type=jnp.float32))
    const = (logp_ref[...] - 0.5 * jnp.sum(logvars, axis=1)[None, :]
             - 0.5 * F_IN * np.log(2.0 * np.pi))
    kidx = lax.broadcasted_iota(jnp.int32, lg.shape, 1)
    lg_ref[...] = jnp.where(kidx < K, lg + const, -1e30)


def _dense2_body(s1a_ref, s1b_ref, s1c_ref, g0a_ref, g0b_ref, g0c_ref,
                 degp_ref, lg_ref, w1_ref, meansT_ref, logvarsT_ref, w2_ref,
                 g1_ref):
    dinv = _dinv(degp_ref[...])
    s1a = s1a_ref[...]
    s1b = s1b_ref[...]
    s1c = s1c_ref[...]
    pa = dinv * (s1a[0] + s1a[1] + g0a_ref[...])
    pm = dinv * jnp.concatenate(
        [s1b[0] + s1b[1] + g0b_ref[...],
         s1c[0] + s1c[1] + g0c_ref[...]], axis=1)
    w1 = w1_ref[...]
    w1sq = w1 * w1
    meansT = meansT_ref[...]
    varT = jnp.exp(logvarsT_ref[...])
    lg = lg_ref[...]
    gmax = jnp.max(lg, axis=1, keepdims=True)
    ge = jnp.exp(lg - gmax)
    gamma = ge / jnp.sum(ge, axis=1, keepdims=True)
    dn = (((1,), (0,)), ((), ()))
    wm = jnp.concatenate([meansT[:, k:k + 1] * w1
                          for k in range(K)], axis=1)
    tx = jax.lax.dot_general(pm, wm, dn, preferred_element_type=jnp.float32)
    wv = jnp.concatenate([varT[:, k:k + 1] * w1sq
                          for k in range(K)], axis=1)
    tc = jax.lax.dot_general(pm, wv, dn, preferred_element_type=jnp.float32)
    pa6 = jnp.concatenate([pa] * K, axis=1)
    tx = tx + pa6
    tce = tc + 1e-8
    rs = lax.rsqrt(tce)
    ratio = tx * rs
    cdf = 0.5 * (1.0 + lax.erf(ratio * np.float32(1.0 / np.sqrt(2.0))))
    pdf = jnp.exp(-0.5 * ratio * ratio) * np.float32(1.0 / np.sqrt(2.0 * np.pi))
    er = tx * cdf + tce * rs * pdf
    kl = lax.broadcasted_iota(jnp.int32, (8, K * HID), 1) // HID
    kr = lax.broadcasted_iota(jnp.int32, (8, K * HID), 0)
    rep = jnp.where(kl == kr, 1.0, 0.0)
    gexp = jax.lax.dot_general(gamma, rep, dn,
                               preferred_element_type=jnp.float32)
    ge_r = gexp * er
    x1 = ge_r[:, 0:HID]
    for k in range(1, K):
        x1 = x1 + ge_r[:, k * HID:(k + 1) * HID]
    g1_ref[...] = dinv * jax.lax.dot_general(x1, w2_ref[...], dn,
                                             preferred_element_type=jnp.float32)


def _dense3_body(s2_ref, g1_ref, degp_ref, b2_ref, w3_ref, x2_ref, g2_ref):
    dinv = _dinv(degp_ref[...])
    s2 = s2_ref[...]
    x2 = jnp.maximum(dinv * (s2[0] + s2[1] + g1_ref[...]) + b2_ref[...], 0.0)
    x2_ref[...] = x2
    g2_ref[...] = dinv * jax.lax.dot_general(
        x2, w3_ref[...], (((1,), (0,)), ((), ())),
        preferred_element_type=jnp.float32)


def _dense4_body(s3_ref, g2_ref, degp_ref, b3_ref, out_ref):
    dinv = _dinv(degp_ref[...])
    s3 = s3_ref[...]
    x3 = dinv * (s3[0] + s3[1] + g2_ref[...]) + b3_ref[...]
    cidx = lax.broadcasted_iota(jnp.int32, x3.shape, 1)
    x3 = jnp.where(cidx < NCLS, x3, -1e30)
    m = jnp.max(x3, axis=1, keepdims=True)
    z = x3 - m
    lse = jnp.log(jnp.sum(jnp.where(cidx < NCLS, jnp.exp(z), 0.0),
                          axis=1, keepdims=True))
    out_ref[...] = (z - lse)[:, :NCLS]


def _row_spec(d):
    return pl.BlockSpec((BLK, d), lambda i: (i, 0))


def _part_spec(d):
    return pl.BlockSpec((NC, BLK, d), lambda i: (0, i, 0))


def _deg_spec():
    return pl.BlockSpec((NC, 1, BLK), lambda i: (0, 0, i))


def _full_spec(shape):
    nd = len(shape)
    return pl.BlockSpec(shape, lambda i: (0,) * nd)


def kernel(x, edge_index, logp, means, logvars, W1, b1, W2, b2, W3, b3):
    f32 = jnp.float32
    pad_i = np.arange(EPAD - E, dtype=np.int32)
    src = jnp.concatenate([edge_index[0], jnp.asarray((pad_i * 97) % N)])
    dst = jnp.concatenate([edge_index[1],
                           jnp.asarray(N + pad_i % (N_ACC - N))])
    src2 = src.reshape(NW * ROWS_PW, CB)
    dst2 = dst.reshape(NW * ROWS_PW, CB)
    means8 = jnp.concatenate([means, jnp.zeros((8 - K, F_IN), f32)], axis=0)
    logvars8 = jnp.concatenate([logvars, jnp.zeros((8 - K, F_IN), f32)], axis=0)
    logp8 = jnp.concatenate([logp, jnp.zeros((8 - K,), f32)]).reshape(1, 8)
    b1r = b1.reshape(1, HID)
    b2r = b2.reshape(1, HID)
    b3r = jnp.concatenate([b3, jnp.zeros((8,), f32)]).reshape(1, NCLS + 8)
    w3p = jnp.concatenate([W3, jnp.zeros((HID, 8), f32)], axis=1)
    z64 = jnp.zeros((CB, HID), f32)
    z48 = jnp.zeros((CB, NCLS + 8), f32)

    grid = N_ACC // BLK
    xp = jnp.pad(x, ((0, N_ACC - N), (0, 0)))

    degp3 = _make_deg()(src2, dst2)

    g0a, g0b, g0c, lg = pl.pallas_call(
        _dense1_body,
        grid=(grid,),
        in_specs=[_row_spec(F_IN), _deg_spec(), _full_spec((F_IN, HID)),
                  _full_spec((1, HID)), _full_spec((8, F_IN)),
                  _full_spec((8, F_IN)), _full_spec((1, 8))],
        out_specs=[_row_spec(HID), _row_spec(HID), _row_spec(HID),
                   _row_spec(8)],
        out_shape=[jax.ShapeDtypeStruct((N_ACC, HID), f32),
                   jax.ShapeDtypeStruct((N_ACC, HID), f32),
                   jax.ShapeDtypeStruct((N_ACC, HID), f32),
                   jax.ShapeDtypeStruct((N_ACC, 8), f32)],
    )(xp, degp3, W1, b1r, means8, logvars8, logp8)

    s1a, s1b, s1c = _make_prop(HID, nt=3)(src2, dst2, g0a, g0b, g0c, z64)

    g1 = pl.pallas_call(
        _dense2_body,
        grid=(grid,),
        in_specs=[_part_spec(HID), _part_spec(HID), _part_spec(HID),
                  _row_spec(HID), _row_spec(HID), _row_spec(HID),
                  _deg_spec(),
                  _row_spec(8), _full_spec((F_IN, HID)), _full_spec((F_IN, 8)),
                  _full_spec((F_IN, 8)), _full_spec((HID, HID))],
        out_specs=[_row_spec(HID)],
        out_shape=[jax.ShapeDtypeStruct((N_ACC, HID), f32)],
    )(s1a, s1b, s1c, g0a, g0b, g0c, degp3, lg, W1, means8.T,
      logvars8.T, W2)[0]

    s2 = _make_prop(HID)(src2, dst2, g1, z64)[0]

    x2, g2 = pl.pallas_call(
        _dense3_body,
        grid=(grid,),
        in_specs=[_part_spec(HID), _row_spec(HID), _deg_spec(),
                  _full_spec((1, HID)), _full_spec((HID, NCLS + 8))],
        out_specs=[_row_spec(HID), _row_spec(NCLS + 8)],
        out_shape=[jax.ShapeDtypeStruct((N_ACC, HID), f32),
                   jax.ShapeDtypeStruct((N_ACC, NCLS + 8), f32)],
    )(s2, g1, degp3, b2r, w3p)

    s3 = _make_prop(NCLS + 8)(src2, dst2, g2, z48)[0]

    out1 = pl.pallas_call(
        _dense4_body,
        grid=(grid,),
        in_specs=[_part_spec(NCLS + 8), _row_spec(NCLS + 8), _deg_spec(),
                  _full_spec((1, NCLS + 8))],
        out_specs=[_row_spec(NCLS)],
        out_shape=[jax.ShapeDtypeStruct((N_ACC, NCLS), f32)],
    )(s3, g2, degp3, b3r)[0]

    return out1[:N], x2[:N]


# speedup vs baseline: 1.0809x; 1.0428x over previous
import functools
import numpy as np
import jax
import jax.numpy as jnp
from jax import lax
from jax.experimental import pallas as pl
from jax.experimental.pallas import tpu as pltpu
from jax.experimental.pallas import tpu_sc as plsc

N = 10000
F_IN = 128
HID = 64
NCLS = 40
K = 6
E = 320000

NC = 2
NS = 16
NW = NC * NS
CB = 128
ROWS_PW = 80
EPAD = NW * ROWS_PW * CB
N_ACC = 10240
TROWS = N_ACC // NS
BLK = 1024


def _worker_id():
    return lax.axis_index("s") * NC + lax.axis_index("c")


def _deg_kernel(src_hbm, dst_hbm, out_hbm, dst_v, ones_v, vbuf, acc_sh, sem):
    c = lax.axis_index("c")
    s = lax.axis_index("s")
    w = _worker_id()
    row0 = s * TROWS
    for i in range(CB // 16):
        ones_v[pl.ds(i * 16, 16)] = jnp.full((16,), 1.0, jnp.float32)
    for i in range(TROWS // 16):
        vbuf[pl.ds(i * 16, 16)] = jnp.zeros((16,), jnp.float32)
    pltpu.sync_copy(vbuf, acc_sh.at[pl.ds(row0, TROWS)])
    plsc.subcore_barrier()
    pltpu.sync_copy(dst_hbm.at[pl.ds(w * ROWS_PW, ROWS_PW)], dst_v)

    def body(j, carry):
        pltpu.sync_copy(ones_v, acc_sh.at[dst_v.at[j]], add=True)
        return carry

    lax.fori_loop(0, ROWS_PW, body, 0)
    plsc.subcore_barrier()
    pltpu.sync_copy(acc_sh.at[pl.ds(row0, TROWS)], vbuf)
    pltpu.sync_copy(vbuf, out_hbm.at[c, 0, pl.ds(row0, TROWS)])


def _ring_pass(g_hbm, src_v, dst_v, bufs, acc_sh, sem):
    b0, b1, b2, b3 = bufs

    def start_g(buf, j):
        pltpu.async_copy(g_hbm.at[src_v.at[j]], buf, sem)

    def wait_g2(x, y):
        pltpu.make_async_copy(g_hbm.at[pl.ds(0, CB)], x, sem).wait()
        pltpu.make_async_copy(g_hbm.at[pl.ds(0, CB)], y, sem).wait()

    def scat(buf, j):
        pltpu.sync_copy(buf, acc_sh.at[dst_v.at[j]], add=True)

    NH = ROWS_PW // 4
    start_g(b0, 0)
    start_g(b1, 1)

    def body(h, carry):
        j = h * 4
        wait_g2(b0, b1)
        start_g(b2, j + 2)
        start_g(b3, j + 3)
        scat(b0, j)
        scat(b1, j + 1)
        wait_g2(b2, b3)

        @pl.when(h + 1 < NH)
        def _():
            start_g(b0, j + 4)
            start_g(b1, j + 5)

        scat(b2, j + 2)
        scat(b3, j + 3)
        return carry

    lax.fori_loop(0, NH, body, 0)


def _prop_kernel(D, nt, src_hbm, dst_hbm, *args):
    tables = args[:nt]
    zeros_hbm = args[nt]
    outs = args[nt + 1:2 * nt + 1]
    src_v, dst_v, b0, b1, b2, b3, acc_sh, sem = args[2 * nt + 1:]
    c = lax.axis_index("c")
    s = lax.axis_index("s")
    w = _worker_id()
    row0 = s * TROWS
    pltpu.sync_copy(zeros_hbm, b0)
    for i in range(TROWS // CB):
        pltpu.sync_copy(b0, acc_sh.at[pl.ds(row0 + i * CB, CB)])
    pltpu.sync_copy(src_hbm.at[pl.ds(w * ROWS_PW, ROWS_PW)], src_v)
    pltpu.sync_copy(dst_hbm.at[pl.ds(w * ROWS_PW, ROWS_PW)], dst_v)
    plsc.subcore_barrier()
    for t in range(nt):
        _ring_pass(tables[t], src_v, dst_v, (b0, b1, b2, b3), acc_sh, sem)
        plsc.subcore_barrier()
        for i in range(TROWS // CB):
            pltpu.sync_copy(acc_sh.at[pl.ds(row0 + i * CB, CB)], b0)
            pltpu.sync_copy(b0, outs[t].at[c, pl.ds(row0 + i * CB, CB)])
        if t + 1 < nt:
            pltpu.sync_copy(zeros_hbm, b0)
            for i in range(TROWS // CB):
                pltpu.sync_copy(b0, acc_sh.at[pl.ds(row0 + i * CB, CB)])
            plsc.subcore_barrier()


def _prop128_kernel(src_hbm, dst_hbm, g_hbm, zeros_hbm, out_hbm,
                    src_v, dst_v, b0, b1, acc_sh, sem0, sem1):
    c = lax.axis_index("c")
    s = lax.axis_index("s")
    w = _worker_id()
    row0 = s * TROWS
    HR = ROWS_PW // 2
    pltpu.sync_copy(zeros_hbm, b0)
    for i in range(TROWS // CB):
        pltpu.sync_copy(b0, acc_sh.at[pl.ds(row0 + i * CB, CB)])
    plsc.subcore_barrier()

    def start_g(buf, sem, j):
        pltpu.async_copy(g_hbm.at[src_v.at[j]], buf, sem)

    def wait_g(buf, sem):
        pltpu.make_async_copy(g_hbm.at[pl.ds(0, CB)], buf, sem).wait()

    def scat(buf, j):
        pltpu.sync_copy(buf, acc_sh.at[dst_v.at[j]], add=True)

    for h in range(2):
        pltpu.sync_copy(src_hbm.at[pl.ds(w * ROWS_PW + h * HR, HR)], src_v)
        pltpu.sync_copy(dst_hbm.at[pl.ds(w * ROWS_PW + h * HR, HR)], dst_v)
        start_g(b0, sem0, 0)

        def body(g, carry):
            j = g * 2
            wait_g(b0, sem0)
            start_g(b1, sem1, j + 1)
            scat(b0, j)
            wait_g(b1, sem1)

            @pl.when(g + 1 < HR // 2)
            def _():
                start_g(b0, sem0, j + 2)

            scat(b1, j + 1)
            return carry

        lax.fori_loop(0, HR // 2, body, 0)
    plsc.subcore_barrier()
    for i in range(TROWS // CB):
        pltpu.sync_copy(acc_sh.at[pl.ds(row0 + i * CB, CB)], b0)
        pltpu.sync_copy(b0, out_hbm.at[c, pl.ds(row0 + i * CB, CB)])


def _make_prop128():
    mesh = plsc.VectorSubcoreMesh(core_axis_name="c", subcore_axis_name="s")
    return functools.partial(
        pl.kernel, _prop128_kernel, mesh=mesh,
        compiler_params=_SC_PARAMS,
        out_type=jax.ShapeDtypeStruct((NC, N_ACC, F_IN), jnp.float32),
        scratch_types=[
            pltpu.VMEM((ROWS_PW // 2, CB), jnp.int32),
            pltpu.VMEM((ROWS_PW // 2, CB), jnp.int32),
            pltpu.VMEM((CB, F_IN), jnp.float32),
            pltpu.VMEM((CB, F_IN), jnp.float32),
            pltpu.VMEM_SHARED((N_ACC, F_IN), jnp.float32),
            pltpu.SemaphoreType.DMA,
            pltpu.SemaphoreType.DMA,
        ],
    )()


_SC_PARAMS = pltpu.CompilerParams(use_tc_tiling_on_sc=False)


def _make_deg():
    mesh = plsc.VectorSubcoreMesh(core_axis_name="c", subcore_axis_name="s")
    return functools.partial(
        pl.kernel, _deg_kernel, mesh=mesh,
        compiler_params=_SC_PARAMS,
        out_type=jax.ShapeDtypeStruct((NC, 1, N_ACC), jnp.float32),
        scratch_types=[
            pltpu.VMEM((ROWS_PW, CB), jnp.int32),
            pltpu.VMEM((CB,), jnp.float32),
            pltpu.VMEM((TROWS,), jnp.float32),
            pltpu.VMEM_SHARED((N_ACC,), jnp.float32),
            pltpu.SemaphoreType.DMA,
        ],
    )()


def _make_prop(D, nt=1):
    mesh = plsc.VectorSubcoreMesh(core_axis_name="c", subcore_axis_name="s")
    return functools.partial(
        pl.kernel, functools.partial(_prop_kernel, D, nt), mesh=mesh,
        compiler_params=_SC_PARAMS,
        out_type=[jax.ShapeDtypeStruct((NC, N_ACC, D), jnp.float32)
                  for _ in range(nt)],
        scratch_types=[
            pltpu.VMEM((ROWS_PW, CB), jnp.int32),
            pltpu.VMEM((ROWS_PW, CB), jnp.int32),
            pltpu.VMEM((CB, D), jnp.float32),
            pltpu.VMEM((CB, D), jnp.float32),
            pltpu.VMEM((CB, D), jnp.float32),
            pltpu.VMEM((CB, D), jnp.float32),
            pltpu.VMEM_SHARED((N_ACC, D), jnp.float32),
            pltpu.SemaphoreType.DMA,
        ],
    )()


def _dinv(degp_v):
    deg = degp_v[0] + degp_v[1] + 1.0
    return lax.rsqrt(jnp.maximum(jnp.transpose(deg, (1, 0)), 1.0))


def _dense1_body(x_ref, degp_ref, w1_ref, b1_ref, means_ref, logvars_ref,
                 logp_ref, t1_ref, t2_ref, lg_ref):
    x = x_ref[...]
    dinv = _dinv(degp_ref[...])
    isn = x != x
    m = jnp.where(isn, 1.0, 0.0)
    xc = jnp.where(isn, 0.0, x)
    a = jax.lax.dot_general(xc, w1_ref[...], (((1,), (0,)), ((), ())),
                            preferred_element_type=jnp.float32) + b1_ref[...]
    t1_ref[...] = dinv * jnp.concatenate([a, m[:, :HID]], axis=1)
    t2_ref[...] = dinv * m[:, HID:]
    means = means_ref[...]
    logvars = logvars_ref[...]
    var = jnp.exp(logvars)
    p1 = -0.5 / var
    p2 = means / var
    p3 = -0.5 * means * means / var
    dn = (((1,), (1,)), ((), ()))
    lg = (jax.lax.dot_general(xc * xc, p1, dn, preferred_element_type=jnp.float32)
          + jax.lax.dot_general(xc, p2, dn, preferred_element_type=jnp.float32)
          + jax.lax.dot_general(1.0 - m, p3, dn, preferred_element_type=jnp.float32))
    const = (logp_ref[...] - 0.5 * jnp.sum(logvars, axis=1)[None, :]
             - 0.5 * F_IN * np.log(2.0 * np.pi))
    kidx = lax.broadcasted_iota(jnp.int32, lg.shape, 1)
    lg_ref[...] = jnp.where(kidx < K, lg + const, -1e30)


def _dense2_body(s1ab_ref, s1c_ref, t1_ref, t2_ref,
                 degp_ref, lg_ref, w1_ref, meansT_ref, logvarsT_ref, w2_ref,
                 g1_ref):
    dinv = _dinv(degp_ref[...])
    s1ab = s1ab_ref[...]
    s1c = s1c_ref[...]
    p128 = dinv * (s1ab[0] + s1ab[1] + t1_ref[...])
    pa = p128[:, :HID]
    pm = jnp.concatenate(
        [p128[:, HID:],
         dinv * (s1c[0] + s1c[1] + t2_ref[...])], axis=1)
    w1 = w1_ref[...]
    w1sq = w1 * w1
    meansT = meansT_ref[...]
    varT = jnp.exp(logvarsT_ref[...])
    lg = lg_ref[...]
    gmax = jnp.max(lg, axis=1, keepdims=True)
    ge = jnp.exp(lg - gmax)
    gamma = ge / jnp.sum(ge, axis=1, keepdims=True)
    dn = (((1,), (0,)), ((), ()))
    wm = jnp.concatenate([meansT[:, k:k + 1] * w1
                          for k in range(K)], axis=1)
    tx = jax.lax.dot_general(pm, wm, dn, preferred_element_type=jnp.float32)
    wv = jnp.concatenate([varT[:, k:k + 1] * w1sq
                          for k in range(K)], axis=1)
    tc = jax.lax.dot_general(pm, wv, dn, preferred_element_type=jnp.float32)
    pa6 = jnp.concatenate([pa] * K, axis=1)
    tx = tx + pa6
    tce = tc + 1e-8
    rs = lax.rsqrt(tce)
    ratio = tx * rs
    cdf = 0.5 * (1.0 + lax.erf(ratio * np.float32(1.0 / np.sqrt(2.0))))
    pdf = jnp.exp(-0.5 * ratio * ratio) * np.float32(1.0 / np.sqrt(2.0 * np.pi))
    er = tx * cdf + tce * rs * pdf
    kl = lax.broadcasted_iota(jnp.int32, (8, K * HID), 1) // HID
    kr = lax.broadcasted_iota(jnp.int32, (8, K * HID), 0)
    rep = jnp.where(kl == kr, 1.0, 0.0)
    gexp = jax.lax.dot_general(gamma, rep, dn,
                               preferred_element_type=jnp.float32)
    ge_r = gexp * er
    x1 = ge_r[:, 0:HID]
    for k in range(1, K):
        x1 = x1 + ge_r[:, k * HID:(k + 1) * HID]
    g1_ref[...] = dinv * jax.lax.dot_general(x1, w2_ref[...], dn,
                                             preferred_element_type=jnp.float32)


def _dense3_body(s2_ref, g1_ref, degp_ref, b2_ref, w3_ref, x2_ref, g2_ref):
    dinv = _dinv(degp_ref[...])
    s2 = s2_ref[...]
    x2 = jnp.maximum(dinv * (s2[0] + s2[1] + g1_ref[...]) + b2_ref[...], 0.0)
    x2_ref[...] = x2
    g2_ref[...] = dinv * jax.lax.dot_general(
        x2, w3_ref[...], (((1,), (0,)), ((), ())),
        preferred_element_type=jnp.float32)


def _dense4_body(s3_ref, g2_ref, degp_ref, b3_ref, out_ref):
    dinv = _dinv(degp_ref[...])
    s3 = s3_ref[...]
    x3 = dinv * (s3[0] + s3[1] + g2_ref[...]) + b3_ref[...]
    cidx = lax.broadcasted_iota(jnp.int32, x3.shape, 1)
    x3 = jnp.where(cidx < NCLS, x3, -1e30)
    m = jnp.max(x3, axis=1, keepdims=True)
    z = x3 - m
    lse = jnp.log(jnp.sum(jnp.where(cidx < NCLS, jnp.exp(z), 0.0),
                          axis=1, keepdims=True))
    out_ref[...] = (z - lse)[:, :NCLS]


def _row_spec(d):
    return pl.BlockSpec((BLK, d), lambda i: (i, 0))


def _part_spec(d):
    return pl.BlockSpec((NC, BLK, d), lambda i: (0, i, 0))


def _deg_spec():
    return pl.BlockSpec((NC, 1, BLK), lambda i: (0, 0, i))


def _full_spec(shape):
    nd = len(shape)
    return pl.BlockSpec(shape, lambda i: (0,) * nd)


def kernel(x, edge_index, logp, means, logvars, W1, b1, W2, b2, W3, b3):
    f32 = jnp.float32
    pad_i = np.arange(EPAD - E, dtype=np.int32)
    src = jnp.concatenate([edge_index[0], jnp.asarray((pad_i * 97) % N)])
    dst = jnp.concatenate([edge_index[1],
                           jnp.asarray(N + pad_i % (N_ACC - N))])
    src2 = src.reshape(NW * ROWS_PW, CB)
    dst2 = dst.reshape(NW * ROWS_PW, CB)
    means8 = jnp.concatenate([means, jnp.zeros((8 - K, F_IN), f32)], axis=0)
    logvars8 = jnp.concatenate([logvars, jnp.zeros((8 - K, F_IN), f32)], axis=0)
    logp8 = jnp.concatenate([logp, jnp.zeros((8 - K,), f32)]).reshape(1, 8)
    b1r = b1.reshape(1, HID)
    b2r = b2.reshape(1, HID)
    b3r = jnp.concatenate([b3, jnp.zeros((8,), f32)]).reshape(1, NCLS + 8)
    w3p = jnp.concatenate([W3, jnp.zeros((HID, 8), f32)], axis=1)
    z128 = jnp.zeros((CB, F_IN), f32)
    z64 = jnp.zeros((CB, HID), f32)
    z48 = jnp.zeros((CB, NCLS + 8), f32)

    grid = N_ACC // BLK
    xp = jnp.pad(x, ((0, N_ACC - N), (0, 0)))

    degp3 = _make_deg()(src2, dst2)

    t1, t2, lg = pl.pallas_call(
        _dense1_body,
        grid=(grid,),
        in_specs=[_row_spec(F_IN), _deg_spec(), _full_spec((F_IN, HID)),
                  _full_spec((1, HID)), _full_spec((8, F_IN)),
                  _full_spec((8, F_IN)), _full_spec((1, 8))],
        out_specs=[_row_spec(F_IN), _row_spec(HID), _row_spec(8)],
        out_shape=[jax.ShapeDtypeStruct((N_ACC, F_IN), f32),
                   jax.ShapeDtypeStruct((N_ACC, HID), f32),
                   jax.ShapeDtypeStruct((N_ACC, 8), f32)],
    )(xp, degp3, W1, b1r, means8, logvars8, logp8)

    s1ab = _make_prop128()(src2, dst2, t1, z128)
    s1c = _make_prop(HID)(src2, dst2, t2, z64)[0]

    g1 = pl.pallas_call(
        _dense2_body,
        grid=(grid,),
        in_specs=[_part_spec(F_IN), _part_spec(HID),
                  _row_spec(F_IN), _row_spec(HID),
                  _deg_spec(),
                  _row_spec(8), _full_spec((F_IN, HID)), _full_spec((F_IN, 8)),
                  _full_spec((F_IN, 8)), _full_spec((HID, HID))],
        out_specs=[_row_spec(HID)],
        out_shape=[jax.ShapeDtypeStruct((N_ACC, HID), f32)],
    )(s1ab, s1c, t1, t2, degp3, lg, W1, means8.T,
      logvars8.T, W2)[0]

    s2 = _make_prop(HID)(src2, dst2, g1, z64)[0]

    x2, g2 = pl.pallas_call(
        _dense3_body,
        grid=(grid,),
        in_specs=[_part_spec(HID), _row_spec(HID), _deg_spec(),
                  _full_spec((1, HID)), _full_spec((HID, NCLS + 8))],
        out_specs=[_row_spec(HID), _row_spec(NCLS + 8)],
        out_shape=[jax.ShapeDtypeStruct((N_ACC, HID), f32),
                   jax.ShapeDtypeStruct((N_ACC, NCLS + 8), f32)],
    )(s2, g1, degp3, b2r, w3p)

    s3 = _make_prop(NCLS + 8)(src2, dst2, g2, z48)[0]

    out1 = pl.pallas_call(
        _dense4_body,
        grid=(grid,),
        in_specs=[_part_spec(NCLS + 8), _row_spec(NCLS + 8), _deg_spec(),
                  _full_spec((1, NCLS + 8))],
        out_specs=[_row_spec(NCLS)],
        out_shape=[jax.ShapeDtypeStruct((N_ACC, NCLS), f32)],
    )(s3, g2, degp3, b3r)[0]

    return out1[:N], x2[:N]


# speedup vs baseline: 1.0816x; 1.0006x over previous
import functools
import numpy as np
import jax
import jax.numpy as jnp
from jax import lax
from jax.experimental import pallas as pl
from jax.experimental.pallas import tpu as pltpu
from jax.experimental.pallas import tpu_sc as plsc

N = 10000
F_IN = 128
HID = 64
NCLS = 40
K = 6
E = 320000

NC = 2
NS = 16
NW = NC * NS
CB = 128
ROWS_PW = 80
EPAD = NW * ROWS_PW * CB
N_ACC = 10240
TROWS = N_ACC // NS
BLK = 1024


def _worker_id():
    return lax.axis_index("s") * NC + lax.axis_index("c")


def _deg_kernel(src_hbm, dst_hbm, out_hbm, dst_v, ones_v, vbuf, acc_sh, sem):
    c = lax.axis_index("c")
    s = lax.axis_index("s")
    w = _worker_id()
    row0 = s * TROWS
    for i in range(CB // 16):
        ones_v[pl.ds(i * 16, 16)] = jnp.full((16,), 1.0, jnp.float32)
    for i in range(TROWS // 16):
        vbuf[pl.ds(i * 16, 16)] = jnp.zeros((16,), jnp.float32)
    pltpu.sync_copy(vbuf, acc_sh.at[pl.ds(row0, TROWS)])
    plsc.subcore_barrier()
    pltpu.sync_copy(dst_hbm.at[pl.ds(w * ROWS_PW, ROWS_PW)], dst_v)

    def body(j, carry):
        pltpu.sync_copy(ones_v, acc_sh.at[dst_v.at[j]], add=True)
        return carry

    lax.fori_loop(0, ROWS_PW, body, 0)
    plsc.subcore_barrier()
    pltpu.sync_copy(acc_sh.at[pl.ds(row0, TROWS)], vbuf)
    pltpu.sync_copy(vbuf, out_hbm.at[c, 0, pl.ds(row0, TROWS)])


def _ring_pass(g_hbm, src_v, dst_v, bufs, acc_sh, sem):
    b0, b1, b2, b3 = bufs

    def start_g(buf, j):
        pltpu.async_copy(g_hbm.at[src_v.at[j]], buf, sem)

    def wait_g2(x, y):
        pltpu.make_async_copy(g_hbm.at[pl.ds(0, CB)], x, sem).wait()
        pltpu.make_async_copy(g_hbm.at[pl.ds(0, CB)], y, sem).wait()

    def scat(buf, j):
        pltpu.sync_copy(buf, acc_sh.at[dst_v.at[j]], add=True)

    NH = ROWS_PW // 4
    start_g(b0, 0)
    start_g(b1, 1)

    def body(h, carry):
        j = h * 4
        wait_g2(b0, b1)
        start_g(b2, j + 2)
        start_g(b3, j + 3)
        scat(b0, j)
        scat(b1, j + 1)
        wait_g2(b2, b3)

        @pl.when(h + 1 < NH)
        def _():
            start_g(b0, j + 4)
            start_g(b1, j + 5)

        scat(b2, j + 2)
        scat(b3, j + 3)
        return carry

    lax.fori_loop(0, NH, body, 0)


def _prop_kernel(D, nt, src_hbm, dst_hbm, *args):
    tables = args[:nt]
    zeros_hbm = args[nt]
    outs = args[nt + 1:2 * nt + 1]
    src_v, dst_v, b0, b1, b2, b3, acc_sh, sem = args[2 * nt + 1:]
    c = lax.axis_index("c")
    s = lax.axis_index("s")
    w = _worker_id()
    row0 = s * TROWS
    pltpu.sync_copy(zeros_hbm, acc_sh.at[pl.ds(row0, TROWS)])
    pltpu.sync_copy(src_hbm.at[pl.ds(w * ROWS_PW, ROWS_PW)], src_v)
    pltpu.sync_copy(dst_hbm.at[pl.ds(w * ROWS_PW, ROWS_PW)], dst_v)
    plsc.subcore_barrier()
    for t in range(nt):
        _ring_pass(tables[t], src_v, dst_v, (b0, b1, b2, b3), acc_sh, sem)
        plsc.subcore_barrier()
        pltpu.sync_copy(acc_sh.at[pl.ds(row0, TROWS)],
                        outs[t].at[c, pl.ds(row0, TROWS)])
        if t + 1 < nt:
            pltpu.sync_copy(zeros_hbm, acc_sh.at[pl.ds(row0, TROWS)])
            plsc.subcore_barrier()


def _prop128_kernel(src_hbm, dst_hbm, g_hbm, zeros_hbm, out_hbm,
                    src_v, dst_v, b0, b1, acc_sh, sem0, sem1):
    c = lax.axis_index("c")
    s = lax.axis_index("s")
    w = _worker_id()
    row0 = s * TROWS
    HR = ROWS_PW // 2
    pltpu.sync_copy(zeros_hbm, acc_sh.at[pl.ds(row0, TROWS)])
    plsc.subcore_barrier()

    def start_g(buf, sem, j):
        pltpu.async_copy(g_hbm.at[src_v.at[j]], buf, sem)

    def wait_g(buf, sem):
        pltpu.make_async_copy(g_hbm.at[pl.ds(0, CB)], buf, sem).wait()

    def scat(buf, j):
        pltpu.sync_copy(buf, acc_sh.at[dst_v.at[j]], add=True)

    for h in range(2):
        pltpu.sync_copy(src_hbm.at[pl.ds(w * ROWS_PW + h * HR, HR)], src_v)
        pltpu.sync_copy(dst_hbm.at[pl.ds(w * ROWS_PW + h * HR, HR)], dst_v)
        start_g(b0, sem0, 0)

        def body(g, carry):
            j = g * 2
            wait_g(b0, sem0)
            start_g(b1, sem1, j + 1)
            scat(b0, j)
            wait_g(b1, sem1)

            @pl.when(g + 1 < HR // 2)
            def _():
                start_g(b0, sem0, j + 2)

            scat(b1, j + 1)
            return carry

        lax.fori_loop(0, HR // 2, body, 0)
    plsc.subcore_barrier()
    pltpu.sync_copy(acc_sh.at[pl.ds(row0, TROWS)],
                    out_hbm.at[c, pl.ds(row0, TROWS)])


def _make_prop128():
    mesh = plsc.VectorSubcoreMesh(core_axis_name="c", subcore_axis_name="s")
    return functools.partial(
        pl.kernel, _prop128_kernel, mesh=mesh,
        compiler_params=_SC_PARAMS,
        out_type=jax.ShapeDtypeStruct((NC, N_ACC, F_IN), jnp.float32),
        scratch_types=[
            pltpu.VMEM((ROWS_PW // 2, CB), jnp.int32),
            pltpu.VMEM((ROWS_PW // 2, CB), jnp.int32),
            pltpu.VMEM((CB, F_IN), jnp.float32),
            pltpu.VMEM((CB, F_IN), jnp.float32),
            pltpu.VMEM_SHARED((N_ACC, F_IN), jnp.float32),
            pltpu.SemaphoreType.DMA,
            pltpu.SemaphoreType.DMA,
        ],
    )()


_SC_PARAMS = pltpu.CompilerParams(use_tc_tiling_on_sc=False)


def _make_deg():
    mesh = plsc.VectorSubcoreMesh(core_axis_name="c", subcore_axis_name="s")
    return functools.partial(
        pl.kernel, _deg_kernel, mesh=mesh,
        compiler_params=_SC_PARAMS,
        out_type=jax.ShapeDtypeStruct((NC, 1, N_ACC), jnp.float32),
        scratch_types=[
            pltpu.VMEM((ROWS_PW, CB), jnp.int32),
            pltpu.VMEM((CB,), jnp.float32),
            pltpu.VMEM((TROWS,), jnp.float32),
            pltpu.VMEM_SHARED((N_ACC,), jnp.float32),
            pltpu.SemaphoreType.DMA,
        ],
    )()


def _make_prop(D, nt=1):
    mesh = plsc.VectorSubcoreMesh(core_axis_name="c", subcore_axis_name="s")
    return functools.partial(
        pl.kernel, functools.partial(_prop_kernel, D, nt), mesh=mesh,
        compiler_params=_SC_PARAMS,
        out_type=[jax.ShapeDtypeStruct((NC, N_ACC, D), jnp.float32)
                  for _ in range(nt)],
        scratch_types=[
            pltpu.VMEM((ROWS_PW, CB), jnp.int32),
            pltpu.VMEM((ROWS_PW, CB), jnp.int32),
            pltpu.VMEM((CB, D), jnp.float32),
            pltpu.VMEM((CB, D), jnp.float32),
            pltpu.VMEM((CB, D), jnp.float32),
            pltpu.VMEM((CB, D), jnp.float32),
            pltpu.VMEM_SHARED((N_ACC, D), jnp.float32),
            pltpu.SemaphoreType.DMA,
        ],
    )()


def _dinv(degp_v):
    deg = degp_v[0] + degp_v[1] + 1.0
    return lax.rsqrt(jnp.maximum(jnp.transpose(deg, (1, 0)), 1.0))


def _dense1_body(x_ref, degp_ref, w1_ref, b1_ref, means_ref, logvars_ref,
                 logp_ref, t1_ref, t2_ref, lg_ref):
    x = x_ref[...]
    dinv = _dinv(degp_ref[...])
    isn = x != x
    m = jnp.where(isn, 1.0, 0.0)
    xc = jnp.where(isn, 0.0, x)
    a = jax.lax.dot_general(xc, w1_ref[...], (((1,), (0,)), ((), ())),
                            preferred_element_type=jnp.float32) + b1_ref[...]
    t1_ref[...] = dinv * jnp.concatenate([a, m[:, :HID]], axis=1)
    t2_ref[...] = dinv * m[:, HID:]
    means = means_ref[...]
    logvars = logvars_ref[...]
    var = jnp.exp(logvars)
    p1 = -0.5 / var
    p2 = means / var
    p3 = -0.5 * means * means / var
    dn = (((1,), (1,)), ((), ()))
    lg = (jax.lax.dot_general(xc * xc, p1, dn, preferred_element_type=jnp.float32)
          + jax.lax.dot_general(xc, p2, dn, preferred_element_type=jnp.float32)
          + jax.lax.dot_general(1.0 - m, p3, dn, preferred_element_type=jnp.float32))
    const = (logp_ref[...] - 0.5 * jnp.sum(logvars, axis=1)[None, :]
             - 0.5 * F_IN * np.log(2.0 * np.pi))
    kidx = lax.broadcasted_iota(jnp.int32, lg.shape, 1)
    lg_ref[...] = jnp.where(kidx < K, lg + const, -1e30)


def _dense2_body(s1ab_ref, s1c_ref, t1_ref, t2_ref,
                 degp_ref, lg_ref, w1_ref, meansT_ref, logvarsT_ref, w2_ref,
                 g1_ref):
    dinv = _dinv(degp_ref[...])
    s1ab = s1ab_ref[...]
    s1c = s1c_ref[...]
    p128 = dinv * (s1ab[0] + s1ab[1] + t1_ref[...])
    pa = p128[:, :HID]
    pm = jnp.concatenate(
        [p128[:, HID:],
         dinv * (s1c[0] + s1c[1] + t2_ref[...])], axis=1)
    w1 = w1_ref[...]
    w1sq = w1 * w1
    meansT = meansT_ref[...]
    varT = jnp.exp(logvarsT_ref[...])
    lg = lg_ref[...]
    gmax = jnp.max(lg, axis=1, keepdims=True)
    ge = jnp.exp(lg - gmax)
    gamma = ge / jnp.sum(ge, axis=1, keepdims=True)
    dn = (((1,), (0,)), ((), ()))
    wm = jnp.concatenate([meansT[:, k:k + 1] * w1
                          for k in range(K)], axis=1)
    tx = jax.lax.dot_general(pm, wm, dn, preferred_element_type=jnp.float32)
    wv = jnp.concatenate([varT[:, k:k + 1] * w1sq
                          for k in range(K)], axis=1)
    tc = jax.lax.dot_general(pm, wv, dn, preferred_element_type=jnp.float32)
    pa6 = jnp.concatenate([pa] * K, axis=1)
    tx = tx + pa6
    tce = tc + 1e-8
    rs = lax.rsqrt(tce)
    ratio = tx * rs
    cdf = 0.5 * (1.0 + lax.erf(ratio * np.float32(1.0 / np.sqrt(2.0))))
    pdf = jnp.exp(-0.5 * ratio * ratio) * np.float32(1.0 / np.sqrt(2.0 * np.pi))
    er = tx * cdf + tce * rs * pdf
    kl = lax.broadcasted_iota(jnp.int32, (8, K * HID), 1) // HID
    kr = lax.broadcasted_iota(jnp.int32, (8, K * HID), 0)
    rep = jnp.where(kl == kr, 1.0, 0.0)
    gexp = jax.lax.dot_general(gamma, rep, dn,
                               preferred_element_type=jnp.float32)
    ge_r = gexp * er
    x1 = ge_r[:, 0:HID]
    for k in range(1, K):
        x1 = x1 + ge_r[:, k * HID:(k + 1) * HID]
    g1_ref[...] = dinv * jax.lax.dot_general(x1, w2_ref[...], dn,
                                             preferred_element_type=jnp.float32)


def _dense3_body(s2_ref, g1_ref, degp_ref, b2_ref, w3_ref, x2_ref, g2_ref):
    dinv = _dinv(degp_ref[...])
    s2 = s2_ref[...]
    x2 = jnp.maximum(dinv * (s2[0] + s2[1] + g1_ref[...]) + b2_ref[...], 0.0)
    x2_ref[...] = x2
    g2_ref[...] = dinv * jax.lax.dot_general(
        x2, w3_ref[...], (((1,), (0,)), ((), ())),
        preferred_element_type=jnp.float32)


def _dense4_body(s3_ref, g2_ref, degp_ref, b3_ref, out_ref):
    dinv = _dinv(degp_ref[...])
    s3 = s3_ref[...]
    x3 = dinv * (s3[0] + s3[1] + g2_ref[...]) + b3_ref[...]
    cidx = lax.broadcasted_iota(jnp.int32, x3.shape, 1)
    x3 = jnp.where(cidx < NCLS, x3, -1e30)
    m = jnp.max(x3, axis=1, keepdims=True)
    z = x3 - m
    lse = jnp.log(jnp.sum(jnp.where(cidx < NCLS, jnp.exp(z), 0.0),
                          axis=1, keepdims=True))
    out_ref[...] = (z - lse)[:, :NCLS]


def _row_spec(d):
    return pl.BlockSpec((BLK, d), lambda i: (i, 0))


def _part_spec(d):
    return pl.BlockSpec((NC, BLK, d), lambda i: (0, i, 0))


def _deg_spec():
    return pl.BlockSpec((NC, 1, BLK), lambda i: (0, 0, i))


def _full_spec(shape):
    nd = len(shape)
    return pl.BlockSpec(shape, lambda i: (0,) * nd)


def kernel(x, edge_index, logp, means, logvars, W1, b1, W2, b2, W3, b3):
    f32 = jnp.float32
    pad_i = np.arange(EPAD - E, dtype=np.int32)
    src = jnp.concatenate([edge_index[0], jnp.asarray((pad_i * 97) % N)])
    dst = jnp.concatenate([edge_index[1],
                           jnp.asarray(N + pad_i % (N_ACC - N))])
    src2 = src.reshape(NW * ROWS_PW, CB)
    dst2 = dst.reshape(NW * ROWS_PW, CB)
    means8 = jnp.concatenate([means, jnp.zeros((8 - K, F_IN), f32)], axis=0)
    logvars8 = jnp.concatenate([logvars, jnp.zeros((8 - K, F_IN), f32)], axis=0)
    logp8 = jnp.concatenate([logp, jnp.zeros((8 - K,), f32)]).reshape(1, 8)
    b1r = b1.reshape(1, HID)
    b2r = b2.reshape(1, HID)
    b3r = jnp.concatenate([b3, jnp.zeros((8,), f32)]).reshape(1, NCLS + 8)
    w3p = jnp.concatenate([W3, jnp.zeros((HID, 8), f32)], axis=1)
    z128 = jnp.zeros((TROWS, F_IN), f32)
    z64 = jnp.zeros((TROWS, HID), f32)
    z48 = jnp.zeros((TROWS, NCLS + 8), f32)

    grid = N_ACC // BLK
    xp = jnp.pad(x, ((0, N_ACC - N), (0, 0)))

    degp3 = _make_deg()(src2, dst2)

    t1, t2, lg = pl.pallas_call(
        _dense1_body,
        grid=(grid,),
        in_specs=[_row_spec(F_IN), _deg_spec(), _full_spec((F_IN, HID)),
                  _full_spec((1, HID)), _full_spec((8, F_IN)),
                  _full_spec((8, F_IN)), _full_spec((1, 8))],
        out_specs=[_row_spec(F_IN), _row_spec(HID), _row_spec(8)],
        out_shape=[jax.ShapeDtypeStruct((N_ACC, F_IN), f32),
                   jax.ShapeDtypeStruct((N_ACC, HID), f32),
                   jax.ShapeDtypeStruct((N_ACC, 8), f32)],
    )(xp, degp3, W1, b1r, means8, logvars8, logp8)

    s1ab = _make_prop128()(src2, dst2, t1, z128)
    s1c = _make_prop(HID)(src2, dst2, t2, z64)[0]

    g1 = pl.pallas_call(
        _dense2_body,
        grid=(grid,),
        in_specs=[_part_spec(F_IN), _part_spec(HID),
                  _row_spec(F_IN), _row_spec(HID),
                  _deg_spec(),
                  _row_spec(8), _full_spec((F_IN, HID)), _full_spec((F_IN, 8)),
                  _full_spec((F_IN, 8)), _full_spec((HID, HID))],
        out_specs=[_row_spec(HID)],
        out_shape=[jax.ShapeDtypeStruct((N_ACC, HID), f32)],
    )(s1ab, s1c, t1, t2, degp3, lg, W1, means8.T,
      logvars8.T, W2)[0]

    s2 = _make_prop(HID)(src2, dst2, g1, z64)[0]

    x2, g2 = pl.pallas_call(
        _dense3_body,
        grid=(grid,),
        in_specs=[_part_spec(HID), _row_spec(HID), _deg_spec(),
                  _full_spec((1, HID)), _full_spec((HID, NCLS + 8))],
        out_specs=[_row_spec(HID), _row_spec(NCLS + 8)],
        out_shape=[jax.ShapeDtypeStruct((N_ACC, HID), f32),
                   jax.ShapeDtypeStruct((N_ACC, NCLS + 8), f32)],
    )(s2, g1, degp3, b2r, w3p)

    s3 = _make_prop(NCLS + 8)(src2, dst2, g2, z48)[0]

    out1 = pl.pallas_call(
        _dense4_body,
        grid=(grid,),
        in_specs=[_part_spec(NCLS + 8), _row_spec(NCLS + 8), _deg_spec(),
                  _full_spec((1, NCLS + 8))],
        out_specs=[_row_spec(NCLS)],
        out_shape=[jax.ShapeDtypeStruct((N_ACC, NCLS), f32)],
    )(s3, g2, degp3, b3r)[0]

    return out1[:N], x2[:N]
